# baseline (speedup 1.0000x reference)
import functools

import jax
import jax.numpy as jnp
from jax import lax
from jax.experimental import pallas as pl
from jax.experimental.pallas import tpu as pltpu
from jax.experimental.pallas import tpu_sc as plsc

N = 100000
D = 20
DH = 16
E = 3200000
NC = 2
NS = 16
NW = NC * NS
EPR = 128
W = 12
ROWS = E // EPR
RPT = 792
ROWS_PAD = RPT * NW
STRIPE = 6248
REM = N - STRIPE * NS
NPAD = N + 8
TRASH = N
CCOL = 4
BLK = 2000
G = N // BLK

_mesh = plsc.VectorSubcoreMesh(
    core_axis_name="c", subcore_axis_name="s", num_cores=NC, num_subcores=NS
)


@functools.partial(
    pl.kernel,
    out_type=jax.ShapeDtypeStruct((NC, N, DH), jnp.float32),
    mesh=_mesh,
    scratch_types=[
        pltpu.VMEM_SHARED((NPAD, DH), jnp.float32),
        pltpu.VMEM((W, EPR), jnp.int32),
        pltpu.VMEM((W, EPR), jnp.int32),
        pltpu.VMEM((W, EPR, DH), jnp.float32),
        pltpu.SemaphoreType.DMA,
        pltpu.SemaphoreType.DMA,
    ],
    compiler_params=pltpu.CompilerParams(use_tc_tiling_on_sc=False),
)
def _segsum(table, srcr, dstr, zf, part_out,
            acc, srcw, dstw, rows, gsem, ssem):
  core = lax.axis_index("c")
  sub = lax.axis_index("s")
  wid = sub * NC + core
  pltpu.sync_copy(zf, acc.at[pl.ds(sub * STRIPE, STRIPE), :])

  @pl.when(sub == 0)
  def _():
    pltpu.sync_copy(zf.at[pl.ds(0, REM + 8), :],
                    acc.at[pl.ds(NS * STRIPE, REM + 8), :])

  plsc.subcore_barrier()
  base = wid * RPT

  def step(g, carry):
    i0 = base + g * W
    pltpu.sync_copy(srcr.at[pl.ds(i0, W)], srcw)
    pltpu.sync_copy(dstr.at[pl.ds(i0, W)], dstw)
    gcps = [pltpu.async_copy(table.at[srcw.at[j]], rows.at[j], gsem)
            for j in range(W)]
    scps = []
    for j in range(W):
      gcps[j].wait()
      scps.append(
          pltpu.async_copy(rows.at[j], acc.at[dstw.at[j]], ssem, add=True))
    for j in range(W):
      scps[j].wait()
    return carry

  lax.fori_loop(0, RPT // W, step, 0)
  plsc.subcore_barrier()
  pltpu.sync_copy(acc.at[pl.ds(sub * STRIPE, STRIPE), :],
                  part_out.at[core, pl.ds(sub * STRIPE, STRIPE), :])

  @pl.when(sub == 0)
  def _():
    pltpu.sync_copy(acc.at[pl.ds(NS * STRIPE, REM), :],
                    part_out.at[core, pl.ds(NS * STRIPE, REM), :])


def _tc1_body(pa_ref, pb_ref, x_ref, wl_ref, wr_ref, b_ref, h_ref, inv_ref):
  sa = pa_ref[0] + pa_ref[1]
  sb = pb_ref[0] + pb_ref[1]
  cnt = sb[:, CCOL:CCOL + 1]
  inv = 1.0 / jnp.maximum(cnt, 1.0)
  m = jnp.concatenate([sa, sb[:, :D - DH]], axis=1) * inv
  h = (jnp.dot(m, wl_ref[...], preferred_element_type=jnp.float32)
       + jnp.dot(x_ref[...], wr_ref[...], preferred_element_type=jnp.float32)
       + b_ref[...])
  h_ref[...] = jnp.maximum(h, 0.0)
  inv_ref[...] = inv


def _tc2_body(pa_ref, pb_ref, inv_ref, h_ref, wl_ref, wr_ref, b_ref, o_ref):
  sa = pa_ref[0] + pa_ref[1]
  sb = pb_ref[0] + pb_ref[1]
  m = jnp.concatenate([sa, sb[:, :D - DH]], axis=1) * inv_ref[...]
  o_ref[...] = (jnp.dot(m, wl_ref[...], preferred_element_type=jnp.float32)
                + jnp.dot(h_ref[...], wr_ref[...],
                          preferred_element_type=jnp.float32)
                + b_ref[...])


_tc1 = pl.pallas_call(
    _tc1_body,
    grid=(G,),
    in_specs=[
        pl.BlockSpec((NC, BLK, DH), lambda i: (0, i, 0)),
        pl.BlockSpec((NC, BLK, DH), lambda i: (0, i, 0)),
        pl.BlockSpec((BLK, D), lambda i: (i, 0)),
        pl.BlockSpec((D, D), lambda i: (0, 0)),
        pl.BlockSpec((D, D), lambda i: (0, 0)),
        pl.BlockSpec((1, D), lambda i: (0, 0)),
    ],
    out_specs=[
        pl.BlockSpec((BLK, D), lambda i: (i, 0)),
        pl.BlockSpec((BLK, 1), lambda i: (i, 0)),
    ],
    out_shape=[
        jax.ShapeDtypeStruct((N, D), jnp.float32),
        jax.ShapeDtypeStruct((N, 1), jnp.float32),
    ],
)

_tc2 = pl.pallas_call(
    _tc2_body,
    grid=(G,),
    in_specs=[
        pl.BlockSpec((NC, BLK, DH), lambda i: (0, i, 0)),
        pl.BlockSpec((NC, BLK, DH), lambda i: (0, i, 0)),
        pl.BlockSpec((BLK, 1), lambda i: (i, 0)),
        pl.BlockSpec((BLK, D), lambda i: (i, 0)),
        pl.BlockSpec((D, D), lambda i: (0, 0)),
        pl.BlockSpec((D, D), lambda i: (0, 0)),
        pl.BlockSpec((1, D), lambda i: (0, 0)),
    ],
    out_specs=pl.BlockSpec((BLK, D), lambda i: (i, 0)),
    out_shape=jax.ShapeDtypeStruct((N, D), jnp.float32),
)


def _halves(t):
  ta = t[:, :DH]
  tb = jnp.concatenate(
      [t[:, DH:], jnp.ones((N, 1), jnp.float32),
       jnp.zeros((N, DH - (D - DH) - 1), jnp.float32)], axis=1)
  return ta, tb


@jax.jit
def kernel(x, edge_index, W1_l, W1_r, b1, W2_l, W2_r, b2):
  ei = edge_index.astype(jnp.int32)
  npad_e = ROWS_PAD * EPR - E
  srcr = jnp.concatenate(
      [ei[0], jnp.zeros((npad_e,), jnp.int32)]).reshape(ROWS_PAD, EPR)
  dstr = jnp.concatenate(
      [ei[1], jnp.full((npad_e,), TRASH, jnp.int32)]).reshape(ROWS_PAD, EPR)
  zf = jnp.zeros((STRIPE, DH), jnp.float32)

  xa, xb = _halves(x)
  p1a = _segsum(xa, srcr, dstr, zf)
  p1b = _segsum(xb, srcr, dstr, zf)
  h, inv = _tc1(p1a, p1b, x, W1_l, W1_r, b1.reshape(1, D))
  ha, hb = _halves(h)
  p2a = _segsum(ha, srcr, dstr, zf)
  p2b = _segsum(hb, srcr, dstr, zf)
  out = _tc2(p2a, p2b, inv, h, W2_l, W2_r, b2.reshape(1, D))
  return out


# speedup vs baseline: 1.1513x; 1.1513x over previous
import functools

import jax
import jax.numpy as jnp
from jax import lax
from jax.experimental import pallas as pl
from jax.experimental.pallas import tpu as pltpu
from jax.experimental.pallas import tpu_sc as plsc

N = 100000
D = 20
DH = 16
E = 3200000
NC = 2
NS = 16
NW = NC * NS
EPR = 128
W = 8
ROWS = E // EPR
RPT = 784
ROWS_PAD = RPT * NW
STRIPE = 6248
REM = N - STRIPE * NS
NPAD = N + 8
TRASH = N
CCOL = 4
BLK = 2000
G = N // BLK

_mesh = plsc.VectorSubcoreMesh(
    core_axis_name="c", subcore_axis_name="s", num_cores=NC, num_subcores=NS
)


@functools.partial(
    pl.kernel,
    out_type=[
        jax.ShapeDtypeStruct((NC, N, DH), jnp.float32),
        jax.ShapeDtypeStruct((NC, N, DH), jnp.float32),
    ],
    mesh=_mesh,
    scratch_types=[
        pltpu.VMEM_SHARED((NPAD, DH), jnp.float32),
        pltpu.VMEM((W, EPR), jnp.int32),
        pltpu.VMEM((W, EPR), jnp.int32),
        pltpu.VMEM((W, EPR, DH), jnp.float32),
        pltpu.SemaphoreType.DMA,
        pltpu.SemaphoreType.DMA,
    ],
    compiler_params=pltpu.CompilerParams(use_tc_tiling_on_sc=False),
)
def _segsum(table_a, table_b, srcr, dstr, zf, out_a, out_b,
            acc, srcw, dstw, rows, gsem, ssem):
  core = lax.axis_index("c")
  sub = lax.axis_index("s")
  wid = sub * NC + core
  base = wid * RPT

  def one_pass(table, part_out):
    pltpu.sync_copy(zf, acc.at[pl.ds(sub * STRIPE, STRIPE), :])

    @pl.when(sub == 0)
    def _():
      pltpu.sync_copy(zf.at[pl.ds(0, REM + 8), :],
                      acc.at[pl.ds(NS * STRIPE, REM + 8), :])

    plsc.subcore_barrier()

    def step(g, carry):
      i0 = base + g * W
      pltpu.sync_copy(srcr.at[pl.ds(i0, W)], srcw)
      pltpu.sync_copy(dstr.at[pl.ds(i0, W)], dstw)
      gcps = [pltpu.async_copy(table.at[srcw.at[j]], rows.at[j], gsem)
              for j in range(W)]
      scps = []
      for j in range(W):
        gcps[j].wait()
        scps.append(
            pltpu.async_copy(rows.at[j], acc.at[dstw.at[j]], ssem, add=True))
      for j in range(W):
        scps[j].wait()
      return carry

    lax.fori_loop(0, RPT // W, step, 0)
    plsc.subcore_barrier()
    pltpu.sync_copy(acc.at[pl.ds(sub * STRIPE, STRIPE), :],
                    part_out.at[core, pl.ds(sub * STRIPE, STRIPE), :])

    @pl.when(sub == 0)
    def _():
      pltpu.sync_copy(acc.at[pl.ds(NS * STRIPE, REM), :],
                      part_out.at[core, pl.ds(NS * STRIPE, REM), :])

    plsc.subcore_barrier()

  one_pass(table_a, out_a)
  one_pass(table_b, out_b)


def _tc1_body(pa_ref, pb_ref, x_ref, wl_ref, wr_ref, b_ref, h_ref, inv_ref):
  sa = pa_ref[0] + pa_ref[1]
  sb = pb_ref[0] + pb_ref[1]
  cnt = sb[:, CCOL:CCOL + 1]
  inv = 1.0 / jnp.maximum(cnt, 1.0)
  m = jnp.concatenate([sa, sb[:, :D - DH]], axis=1) * inv
  h = (jnp.dot(m, wl_ref[...], preferred_element_type=jnp.float32)
       + jnp.dot(x_ref[...], wr_ref[...], preferred_element_type=jnp.float32)
       + b_ref[...])
  h_ref[...] = jnp.maximum(h, 0.0)
  inv_ref[...] = inv


def _tc2_body(pa_ref, pb_ref, inv_ref, h_ref, wl_ref, wr_ref, b_ref, o_ref):
  sa = pa_ref[0] + pa_ref[1]
  sb = pb_ref[0] + pb_ref[1]
  m = jnp.concatenate([sa, sb[:, :D - DH]], axis=1) * inv_ref[...]
  o_ref[...] = (jnp.dot(m, wl_ref[...], preferred_element_type=jnp.float32)
                + jnp.dot(h_ref[...], wr_ref[...],
                          preferred_element_type=jnp.float32)
                + b_ref[...])


_tc1 = pl.pallas_call(
    _tc1_body,
    grid=(G,),
    in_specs=[
        pl.BlockSpec((NC, BLK, DH), lambda i: (0, i, 0)),
        pl.BlockSpec((NC, BLK, DH), lambda i: (0, i, 0)),
        pl.BlockSpec((BLK, D), lambda i: (i, 0)),
        pl.BlockSpec((D, D), lambda i: (0, 0)),
        pl.BlockSpec((D, D), lambda i: (0, 0)),
        pl.BlockSpec((1, D), lambda i: (0, 0)),
    ],
    out_specs=[
        pl.BlockSpec((BLK, D), lambda i: (i, 0)),
        pl.BlockSpec((BLK, 1), lambda i: (i, 0)),
    ],
    out_shape=[
        jax.ShapeDtypeStruct((N, D), jnp.float32),
        jax.ShapeDtypeStruct((N, 1), jnp.float32),
    ],
)

_tc2 = pl.pallas_call(
    _tc2_body,
    grid=(G,),
    in_specs=[
        pl.BlockSpec((NC, BLK, DH), lambda i: (0, i, 0)),
        pl.BlockSpec((NC, BLK, DH), lambda i: (0, i, 0)),
        pl.BlockSpec((BLK, 1), lambda i: (i, 0)),
        pl.BlockSpec((BLK, D), lambda i: (i, 0)),
        pl.BlockSpec((D, D), lambda i: (0, 0)),
        pl.BlockSpec((D, D), lambda i: (0, 0)),
        pl.BlockSpec((1, D), lambda i: (0, 0)),
    ],
    out_specs=pl.BlockSpec((BLK, D), lambda i: (i, 0)),
    out_shape=jax.ShapeDtypeStruct((N, D), jnp.float32),
)


def _halves(t):
  ta = t[:, :DH]
  tb = jnp.concatenate(
      [t[:, DH:], jnp.ones((N, 1), jnp.float32),
       jnp.zeros((N, DH - (D - DH) - 1), jnp.float32)], axis=1)
  return ta, tb


@jax.jit
def kernel(x, edge_index, W1_l, W1_r, b1, W2_l, W2_r, b2):
  ei = edge_index.astype(jnp.int32)
  npad_e = ROWS_PAD * EPR - E
  srcr = jnp.concatenate(
      [ei[0], jnp.zeros((npad_e,), jnp.int32)]).reshape(ROWS_PAD, EPR)
  dstr = jnp.concatenate(
      [ei[1], jnp.full((npad_e,), TRASH, jnp.int32)]).reshape(ROWS_PAD, EPR)
  zf = jnp.zeros((STRIPE, DH), jnp.float32)

  xa, xb = _halves(x)
  p1a, p1b = _segsum(xa, xb, srcr, dstr, zf)
  h, inv = _tc1(p1a, p1b, x, W1_l, W1_r, b1.reshape(1, D))
  ha, hb = _halves(h)
  p2a, p2b = _segsum(ha, hb, srcr, dstr, zf)
  out = _tc2(p2a, p2b, inv, h, W2_l, W2_r, b2.reshape(1, D))
  return out


# speedup vs baseline: 1.2821x; 1.1136x over previous
import functools

import jax
import jax.numpy as jnp
from jax import lax
from jax.experimental import pallas as pl
from jax.experimental.pallas import tpu as pltpu
from jax.experimental.pallas import tpu_sc as plsc

N = 100000
D = 20
DH = 16
E = 3200000
NC = 2
NS = 16
NW = NC * NS
EPR = 128
W = 8
ROWS = E // EPR
RPT = 1568
ROWS_PAD = RPT * NS
STRIPE = 6248
REM = N - STRIPE * NS
NPAD = N + 8
TRASH = N
CCOL = 4
BLK = 2000
G = N // BLK

_mesh = plsc.VectorSubcoreMesh(
    core_axis_name="c", subcore_axis_name="s", num_cores=NC, num_subcores=NS
)


@functools.partial(
    pl.kernel,
    out_type=jax.ShapeDtypeStruct((NC, N, DH), jnp.float32),
    mesh=_mesh,
    scratch_types=[
        pltpu.VMEM_SHARED((NPAD, DH), jnp.float32),
        pltpu.VMEM((W, EPR), jnp.int32),
        pltpu.VMEM((W, EPR), jnp.int32),
        pltpu.VMEM((W, EPR, DH), jnp.float32),
        pltpu.SemaphoreType.DMA,
        pltpu.SemaphoreType.DMA,
    ],
    compiler_params=pltpu.CompilerParams(use_tc_tiling_on_sc=False),
)
def _segsum(tab2, srcr, dstr, zf, out2,
            acc, srcw, dstw, rows, gsem, ssem):
  core = lax.axis_index("c")
  sub = lax.axis_index("s")
  base = sub * RPT
  table = tab2.at[core]
  pltpu.sync_copy(zf, acc.at[pl.ds(sub * STRIPE, STRIPE), :])

  @pl.when(sub == 0)
  def _():
    pltpu.sync_copy(zf.at[pl.ds(0, REM + 8), :],
                    acc.at[pl.ds(NS * STRIPE, REM + 8), :])

  plsc.subcore_barrier()

  def step(g, carry):
    i0 = base + g * W
    pltpu.sync_copy(srcr.at[pl.ds(i0, W)], srcw)
    pltpu.sync_copy(dstr.at[pl.ds(i0, W)], dstw)
    gcps = [pltpu.async_copy(table.at[srcw.at[j]], rows.at[j], gsem)
            for j in range(W)]
    scps = []
    for j in range(W):
      gcps[j].wait()
      scps.append(
          pltpu.async_copy(rows.at[j], acc.at[dstw.at[j]], ssem, add=True))
    for j in range(W):
      scps[j].wait()
    return carry

  lax.fori_loop(0, RPT // W, step, 0)
  plsc.subcore_barrier()
  pltpu.sync_copy(acc.at[pl.ds(sub * STRIPE, STRIPE), :],
                  out2.at[core, pl.ds(sub * STRIPE, STRIPE), :])

  @pl.when(sub == 0)
  def _():
    pltpu.sync_copy(acc.at[pl.ds(NS * STRIPE, REM), :],
                    out2.at[core, pl.ds(NS * STRIPE, REM), :])


def _tc1_body(p_ref, x_ref, wl_ref, wr_ref, b_ref, h_ref, inv_ref):
  sa = p_ref[0]
  sb = p_ref[1]
  cnt = sb[:, CCOL:CCOL + 1]
  inv = 1.0 / jnp.maximum(cnt, 1.0)
  m = jnp.concatenate([sa, sb[:, :D - DH]], axis=1) * inv
  h = (jnp.dot(m, wl_ref[...], preferred_element_type=jnp.float32)
       + jnp.dot(x_ref[...], wr_ref[...], preferred_element_type=jnp.float32)
       + b_ref[...])
  h_ref[...] = jnp.maximum(h, 0.0)
  inv_ref[...] = inv


def _tc2_body(p_ref, inv_ref, h_ref, wl_ref, wr_ref, b_ref, o_ref):
  sa = p_ref[0]
  sb = p_ref[1]
  m = jnp.concatenate([sa, sb[:, :D - DH]], axis=1) * inv_ref[...]
  o_ref[...] = (jnp.dot(m, wl_ref[...], preferred_element_type=jnp.float32)
                + jnp.dot(h_ref[...], wr_ref[...],
                          preferred_element_type=jnp.float32)
                + b_ref[...])


_tc1 = pl.pallas_call(
    _tc1_body,
    grid=(G,),
    in_specs=[
        pl.BlockSpec((NC, BLK, DH), lambda i: (0, i, 0)),
        pl.BlockSpec((BLK, D), lambda i: (i, 0)),
        pl.BlockSpec((D, D), lambda i: (0, 0)),
        pl.BlockSpec((D, D), lambda i: (0, 0)),
        pl.BlockSpec((1, D), lambda i: (0, 0)),
    ],
    out_specs=[
        pl.BlockSpec((BLK, D), lambda i: (i, 0)),
        pl.BlockSpec((BLK, 1), lambda i: (i, 0)),
    ],
    out_shape=[
        jax.ShapeDtypeStruct((N, D), jnp.float32),
        jax.ShapeDtypeStruct((N, 1), jnp.float32),
    ],
)

_tc2 = pl.pallas_call(
    _tc2_body,
    grid=(G,),
    in_specs=[
        pl.BlockSpec((NC, BLK, DH), lambda i: (0, i, 0)),
        pl.BlockSpec((BLK, 1), lambda i: (i, 0)),
        pl.BlockSpec((BLK, D), lambda i: (i, 0)),
        pl.BlockSpec((D, D), lambda i: (0, 0)),
        pl.BlockSpec((D, D), lambda i: (0, 0)),
        pl.BlockSpec((1, D), lambda i: (0, 0)),
    ],
    out_specs=pl.BlockSpec((BLK, D), lambda i: (i, 0)),
    out_shape=jax.ShapeDtypeStruct((N, D), jnp.float32),
)


def _halves(t):
  ta = t[:, :DH]
  tb = jnp.concatenate(
      [t[:, DH:], jnp.ones((N, 1), jnp.float32),
       jnp.zeros((N, DH - (D - DH) - 1), jnp.float32)], axis=1)
  return ta, tb


@jax.jit
def kernel(x, edge_index, W1_l, W1_r, b1, W2_l, W2_r, b2):
  ei = edge_index.astype(jnp.int32)
  npad_e = ROWS_PAD * EPR - E
  srcr = jnp.concatenate(
      [ei[0], jnp.zeros((npad_e,), jnp.int32)]).reshape(ROWS_PAD, EPR)
  dstr = jnp.concatenate(
      [ei[1], jnp.full((npad_e,), TRASH, jnp.int32)]).reshape(ROWS_PAD, EPR)
  zf = jnp.zeros((STRIPE, DH), jnp.float32)

  xa, xb = _halves(x)
  p1 = _segsum(jnp.stack([xa, xb]), srcr, dstr, zf)
  h, inv = _tc1(p1, x, W1_l, W1_r, b1.reshape(1, D))
  ha, hb = _halves(h)
  p2 = _segsum(jnp.stack([ha, hb]), srcr, dstr, zf)
  out = _tc2(p2, inv, h, W2_l, W2_r, b2.reshape(1, D))
  return out
